# Initial kernel scaffold; baseline (speedup 1.0000x reference)
import jax
import jax.numpy as jnp
from jax.experimental import pallas as pl


def kernel(x, edge_index, batch, params):
    raise NotImplementedError("write your pallas kernel here")


# speedup vs baseline: 1.0001x; 1.0001x over previous
import functools

import jax
import jax.numpy as jnp
from jax.experimental import pallas as pl

N = 100000
E = 1600000
G = 16
H = 4
C = 32

_POOL_BLK = 1000


def _pool_body(batch_ref, hf_ref, hb_ref, hn_ref, acc_ref):
    i = pl.program_id(0)
    hf = hf_ref[...]
    hb = hb_ref[...]
    hn = jnp.concatenate([hf, hb], axis=1)
    hn_ref[...] = hn
    b = batch_ref[0, 0, :]
    onehot = (b[:, None] == jax.lax.broadcasted_iota(jnp.int32, (1, G), 1)
              ).astype(jnp.float32)
    hext = jnp.concatenate(
        [hn, jnp.ones((_POOL_BLK, 1), jnp.float32),
         jnp.zeros((_POOL_BLK, 128 - 2 * C - 1), jnp.float32)], axis=1)
    part = jax.lax.dot_general(onehot, hext, (((0,), (0,)), ((), ())),
                               preferred_element_type=jnp.float32)

    @pl.when(i == 0)
    def _():
        acc_ref[...] = jnp.zeros_like(acc_ref)

    acc_ref[...] += part


def _concat_pool(hf, hb, batch):
    nb = N // _POOL_BLK
    batch3 = batch.reshape(nb, 1, _POOL_BLK)
    hn, acc = pl.pallas_call(
        _pool_body,
        grid=(nb,),
        in_specs=[
            pl.BlockSpec((1, 1, _POOL_BLK), lambda i: (i, 0, 0)),
            pl.BlockSpec((_POOL_BLK, C), lambda i: (i, 0)),
            pl.BlockSpec((_POOL_BLK, C), lambda i: (i, 0)),
        ],
        out_specs=[
            pl.BlockSpec((_POOL_BLK, 2 * C), lambda i: (i, 0)),
            pl.BlockSpec((G, 128), lambda i: (0, 0)),
        ],
        out_shape=[
            jax.ShapeDtypeStruct((N, 2 * C), jnp.float32),
            jax.ShapeDtypeStruct((G, 128), jnp.float32),
        ],
    )(batch3, hf, hb)
    g_pool = acc[:, :2 * C] / jnp.clip(acc[:, 2 * C:2 * C + 1], 1.0)
    return hn, g_pool


def _gat_layer(x, p, src, dst, n, heads, out_ch, concat):
    xp = (x @ p['W']).reshape(n, heads, out_ch)
    a_src = (xp * p['as'][None]).sum(-1)
    a_dst = (xp * p['ad'][None]).sum(-1)
    e = jax.nn.leaky_relu(a_src[src] + a_dst[dst], 0.2)
    m = jax.ops.segment_max(e, dst, num_segments=n)
    ex = jnp.exp(e - m[dst])
    den = jax.ops.segment_sum(ex, dst, num_segments=n)
    alpha = ex / (den[dst] + 1e-16)
    out = jax.ops.segment_sum(alpha[:, :, None] * xp[src], dst, num_segments=n)
    out = out.reshape(n, heads * out_ch) if concat else out.mean(axis=1)
    return out + p['b']


def kernel(x, edge_index, batch, params):
    n = x.shape[0]
    ar = jnp.arange(n, dtype=edge_index.dtype)
    ei = jnp.concatenate([edge_index, jnp.stack([ar, ar])], axis=1)
    s, d = ei[0], ei[1]
    hf = x[:, jnp.array([0, 1, 3])]
    hf = jax.nn.elu(_gat_layer(hf, params['f1'], s, d, n, H, C, True))
    hf = jax.nn.elu(_gat_layer(hf, params['f2'], s, d, n, H, C, True))
    hf = _gat_layer(hf, params['f3'], s, d, n, 1, C, False)
    hb = x[:, jnp.array([0, 2, 4])]
    hb = jax.nn.elu(_gat_layer(hb, params['b1'], d, s, n, H, C, True))
    hb = jax.nn.elu(_gat_layer(hb, params['b2'], d, s, n, H, C, True))
    hb = _gat_layer(hb, params['b3'], d, s, n, 1, C, False)
    return _concat_pool(hf, hb, batch)


# speedup vs baseline: 37.9815x; 37.9762x over previous
import functools

import jax
import jax.numpy as jnp
from jax import lax
from jax.experimental import pallas as pl
from jax.experimental.pallas import tpu as pltpu
from jax.experimental.pallas import tpu_sc as plsc

N = 100000
E = 1600000
G = 16
H = 4
C = 32
HC = H * C
R4 = 256
R1 = 128
BLK = 1000
NB = N // BLK
NEG = -1e30


def _prep_first_body(h_ref, w_ref, am_ref, t_ref, ad_ref, b_ref):
    i = pl.program_id(0)
    xp = jnp.dot(h_ref[...], w_ref[...], preferred_element_type=jnp.float32)
    av = jnp.dot(xp, am_ref[...], preferred_element_type=jnp.float32)
    t_ref[...] = jnp.concatenate(
        [xp, av[:, :H], jnp.zeros((BLK, R4 - HC - H), jnp.float32)], axis=1)
    ad_ref[...] = av[:, H:2 * H]
    cm = jnp.max(av, axis=0, keepdims=True)
    contrib = jnp.concatenate(
        [jnp.broadcast_to(cm, (8, 2 * H)), jnp.full((8, 128 - 2 * H), NEG)], axis=1)

    @pl.when(i == 0)
    def _():
        b_ref[...] = jnp.full((8, 128), NEG, jnp.float32)

    b_ref[...] = jnp.maximum(b_ref[...], contrib)


def _prep_mid_body(acc_ref, bias_ref, dsel_ref, w_ref, am_ref,
                   t_ref, ad_ref, b_ref):
    i = pl.program_id(0)
    acc = acc_ref[...]
    den = jnp.dot(acc, dsel_ref[...], preferred_element_type=jnp.float32)
    h = acc[:, :HC] / den + bias_ref[...]
    h = jnp.where(h > 0, h, jnp.exp(h) - 1.0)
    xp = jnp.dot(h, w_ref[...], preferred_element_type=jnp.float32)
    av = jnp.dot(xp, am_ref[...], preferred_element_type=jnp.float32)
    nout = w_ref.shape[1]
    t_ref[...] = jnp.concatenate(
        [xp, av[:, :1] if nout == C else av[:, :H],
         jnp.zeros((BLK, (R1 - C - 1) if nout == C else (R4 - HC - H)),
                   jnp.float32)], axis=1)
    ad_ref[...] = av[:, H:2 * H] if nout == HC else jnp.concatenate(
        [av[:, 4:5], jnp.zeros((BLK, 3), jnp.float32)], axis=1)
    cm = jnp.max(av, axis=0, keepdims=True)
    contrib = jnp.concatenate(
        [jnp.broadcast_to(cm, (8, 2 * H)), jnp.full((8, 128 - 2 * H), NEG)], axis=1)

    @pl.when(i == 0)
    def _():
        b_ref[...] = jnp.full((8, 128), NEG, jnp.float32)

    b_ref[...] = jnp.maximum(b_ref[...], contrib)


def _run_prep_first(hcols, W, AM):
    return pl.pallas_call(
        _prep_first_body,
        grid=(NB,),
        in_specs=[
            pl.BlockSpec((BLK, 3), lambda i: (i, 0)),
            pl.BlockSpec((3, HC), lambda i: (0, 0)),
            pl.BlockSpec((HC, 8), lambda i: (0, 0)),
        ],
        out_specs=[
            pl.BlockSpec((BLK, R4), lambda i: (i, 0)),
            pl.BlockSpec((BLK, 4), lambda i: (i, 0)),
            pl.BlockSpec((8, 128), lambda i: (0, 0)),
        ],
        out_shape=[
            jax.ShapeDtypeStruct((N, R4), jnp.float32),
            jax.ShapeDtypeStruct((N, 4), jnp.float32),
            jax.ShapeDtypeStruct((8, 128), jnp.float32),
        ],
    )(hcols, W, AM)


def _run_prep_mid(ACC, bias, DSEL, W, AM):
    nout = W.shape[1]
    rout = R4 if nout == HC else R1
    return pl.pallas_call(
        _prep_mid_body,
        grid=(NB,),
        in_specs=[
            pl.BlockSpec((BLK, R4), lambda i: (i, 0)),
            pl.BlockSpec((1, HC), lambda i: (0, 0)),
            pl.BlockSpec((R4, HC), lambda i: (0, 0)),
            pl.BlockSpec((HC, nout), lambda i: (0, 0)),
            pl.BlockSpec((nout, 8), lambda i: (0, 0)),
        ],
        out_specs=[
            pl.BlockSpec((BLK, rout), lambda i: (i, 0)),
            pl.BlockSpec((BLK, 4), lambda i: (i, 0)),
            pl.BlockSpec((8, 128), lambda i: (0, 0)),
        ],
        out_shape=[
            jax.ShapeDtypeStruct((N, rout), jnp.float32),
            jax.ShapeDtypeStruct((N, 4), jnp.float32),
            jax.ShapeDtypeStruct((8, 128), jnp.float32),
        ],
    )(ACC, bias, DSEL, W, AM)


def _pool_body(batch_ref, af_ref, ab_ref, bf_ref, bb_ref, hn_ref, acc_ref):
    i = pl.program_id(0)
    af = af_ref[...]
    ab = ab_ref[...]
    hf = af[:, :C] / af[:, C:C + 1] + bf_ref[...]
    hb = ab[:, :C] / ab[:, C:C + 1] + bb_ref[...]
    hn = jnp.concatenate([hf, hb], axis=1)
    hn_ref[...] = hn
    b = batch_ref[0, 0, :]
    onehot = (b[:, None] == lax.broadcasted_iota(jnp.int32, (1, G), 1)
              ).astype(jnp.float32)
    hext = jnp.concatenate(
        [hn, jnp.ones((BLK, 1), jnp.float32),
         jnp.zeros((BLK, 128 - 2 * C - 1), jnp.float32)], axis=1)
    part = jax.lax.dot_general(onehot, hext, (((0,), (0,)), ((), ())),
                               preferred_element_type=jnp.float32)

    @pl.when(i == 0)
    def _():
        acc_ref[...] = jnp.zeros_like(acc_ref)

    acc_ref[...] += part


def _run_pool(ACCf, ACCb, bf, bb, batch):
    batch3 = batch.reshape(NB, 1, BLK)
    return pl.pallas_call(
        _pool_body,
        grid=(NB,),
        in_specs=[
            pl.BlockSpec((1, 1, BLK), lambda i: (i, 0, 0)),
            pl.BlockSpec((BLK, R1), lambda i: (i, 0)),
            pl.BlockSpec((BLK, R1), lambda i: (i, 0)),
            pl.BlockSpec((1, C), lambda i: (0, 0)),
            pl.BlockSpec((1, C), lambda i: (0, 0)),
        ],
        out_specs=[
            pl.BlockSpec((BLK, 2 * C), lambda i: (i, 0)),
            pl.BlockSpec((G, 128), lambda i: (0, 0)),
        ],
        out_shape=[
            jax.ShapeDtypeStruct((N, 2 * C), jnp.float32),
            jax.ShapeDtypeStruct((G, 128), jnp.float32),
        ],
    )(batch3, ACCf, ACCb, bf, bb)


EP = E + N
CN = 256
NCHUNK = -(-N // CN)
NP = NCHUNK * CN
ACCROWS = CN + 16
BE = 64
NW = 32
EPP = EP + BE


def _sload(ref, i):
    v = plsc.load_gather(ref, [jnp.full((16,), i, jnp.int32)])
    return jnp.max(v)


def _edge_agg_sc_body(heads, t_h, ad_h, b_h, src_h, dst_h, es_h, out_h,
                      acc_v, rows_v, srcb_v, dstb_v, sidx_v, ad_v, brep_v,
                      es_v, sem):
    hc = heads * C
    rw = R4 if heads == H else R1
    cid = lax.axis_index("c")
    sid = lax.axis_index("s")
    wid = cid * 16 + sid
    lane = lax.iota(jnp.int32, 16)
    pltpu.sync_copy(b_h, brep_v)
    pltpu.sync_copy(es_h, es_v)
    nfull = NCHUNK // NW
    nck = jnp.where(wid < NCHUNK - NW * nfull, nfull + 1, nfull)

    def chunk_body(k, carry):
        ci = wid + NW * k
        d0 = ci * CN
        zero16 = jnp.zeros((16,), jnp.float32)

        def zero_body(i, zc):
            for jj in range(rw // 16):
                acc_v[i, pl.ds(jj * 16, 16)] = zero16
            return zc

        lax.fori_loop(0, ACCROWS, zero_body, 0)
        pltpu.sync_copy(ad_h.at[pl.ds(d0, CN)], ad_v.at[pl.ds(0, CN)])
        e0 = _sload(es_v, ci)
        e1 = _sload(es_v, ci + 1)
        base0 = (e0 // 8) * 8
        nbatch = (e1 - base0 + BE - 1) // BE

        def batch_body(j, bc):
            base = base0 + j * BE
            pltpu.sync_copy(src_h.at[pl.ds(base, BE)], srcb_v)
            pltpu.sync_copy(dst_h.at[pl.ds(base, BE)], dstb_v)
            pltpu.async_copy(t_h.at[srcb_v], rows_v, sem).wait()
            for g in range(BE // 16):
                eid = base + g * 16 + lane
                valid = (eid >= e0) & (eid < e1)
                dst16 = dstb_v[pl.ds(g * 16, 16)]
                dl = jnp.where(valid, dst16 - d0, CN)
                sidx_v[pl.ds(g * 16, 16)] = dl
                eloc = g * 16 + lane
                for h in range(heads):
                    colv = jnp.full((16,), hc + h, jnp.int32)
                    a_s = plsc.load_gather(rows_v, [eloc, colv])
                    a_d = plsc.load_gather(ad_v, [dl, jnp.full((16,), h, jnp.int32)])
                    lg = a_s + a_d
                    lg = jnp.where(lg > 0, lg, 0.2 * lg)
                    w = jnp.exp(lg - brep_v[h])
                    w = jnp.where(valid, w, 0.0)
                    plsc.store_scatter(rows_v, [eloc, colv], w)
            def edge_body(e, ec):
                esp16 = jnp.full((16,), e, jnp.int32)
                dlsp = plsc.load_gather(sidx_v, [esp16])
                for h in range(heads):
                    wsp = plsc.load_gather(
                        rows_v, [esp16, jnp.full((16,), hc + h, jnp.int32)])
                    for q in range(C // 16):
                        cc = h * C + q * 16
                        plsc.addupdate_scatter(
                            acc_v, [dlsp, cc + lane],
                            rows_v[e, pl.ds(cc, 16)] * wsp)
                plsc.addupdate_scatter(
                    acc_v, [dlsp, hc + lane], rows_v[e, pl.ds(hc, 16)])
                return ec

            lax.fori_loop(0, BE, edge_body, 0)
            return bc

        lax.fori_loop(0, nbatch, batch_body, 0)
        pltpu.sync_copy(acc_v.at[pl.ds(0, CN)], out_h.at[pl.ds(d0, CN)])
        return carry

    lax.fori_loop(0, nck, chunk_body, 0)


def _edge_agg(T, AD, B, srcp, dstp, esp, heads):
    r = R4 if heads == H else R1
    AD = jnp.pad(AD, ((0, NP - N), (0, 0)))
    brep = jnp.pad(B, (0, 8 - heads))[:, None] * jnp.ones((1, 16), jnp.float32)
    mesh = plsc.VectorSubcoreMesh(core_axis_name="c", subcore_axis_name="s")
    fn = pl.kernel(
        functools.partial(_edge_agg_sc_body, heads),
        mesh=mesh,
        out_type=jax.ShapeDtypeStruct((NP, r), jnp.float32),
        compiler_params=pltpu.CompilerParams(needs_layout_passes=False),
        scratch_types=[
            pltpu.VMEM((ACCROWS, r), jnp.float32),
            pltpu.VMEM((BE, r), jnp.float32),
            pltpu.VMEM((BE,), jnp.int32),
            pltpu.VMEM((BE,), jnp.int32),
            pltpu.VMEM((BE,), jnp.int32),
            pltpu.VMEM((ACCROWS, 4), jnp.float32),
            pltpu.VMEM((8, 16), jnp.float32),
            pltpu.VMEM((400,), jnp.int32),
            pltpu.SemaphoreType.DMA,
        ],
    )
    return fn(T, AD, brep, srcp, dstp, esp)[:N]


def _mk_am(p, heads):
    nout = heads * C
    am = jnp.zeros((nout, 8), jnp.float32)
    for h in range(heads):
        am = am.at[h * C:(h + 1) * C, h].set(p['as'][h])
        am = am.at[h * C:(h + 1) * C, 4 + h].set(p['ad'][h])
    return am


def _bvec(bacc, heads):
    b = bacc[0, :heads] + bacc[0, 4:4 + heads]
    b = jnp.where(b > 0, b, 0.2 * b)
    return b


def _sorted_edges(src, dst):
    perm = jnp.argsort(dst)
    srcs = src[perm]
    dsts = dst[perm]
    es = jnp.searchsorted(dsts, jnp.arange(NCHUNK + 1) * CN).astype(jnp.int32)
    esp = jnp.pad(es, (0, 400 - NCHUNK - 1), constant_values=EP)
    srcp = jnp.pad(srcs, (0, BE))
    dstp = jnp.pad(dsts, (0, BE))
    return srcp, dstp, esp


def _direction(hcols, edges, p1, p2, p3, dsel):
    srcp, dstp, esp = edges
    t, ad, bacc = _run_prep_first(hcols, p1['W'], _mk_am(p1, H))
    acc = _edge_agg(t, ad, _bvec(bacc, H), srcp, dstp, esp, H)
    t, ad, bacc = _run_prep_mid(acc, p1['b'].reshape(1, HC), dsel,
                                p2['W'], _mk_am(p2, H))
    acc = _edge_agg(t, ad, _bvec(bacc, H), srcp, dstp, esp, H)
    am3 = jnp.zeros((C, 8), jnp.float32)
    am3 = am3.at[:, 0].set(p3['as'][0]).at[:, 4].set(p3['ad'][0])
    t, ad, bacc = _run_prep_mid(acc, p2['b'].reshape(1, HC), dsel,
                                p3['W'], am3)
    acc = _edge_agg(t, ad, _bvec(bacc, 1), srcp, dstp, esp, 1)
    return acc


def kernel(x, edge_index, batch, params):
    ar = jnp.arange(N, dtype=edge_index.dtype)
    ei = jnp.concatenate([edge_index, jnp.stack([ar, ar])], axis=1)
    s, d = ei[0], ei[1]
    edges_f = _sorted_edges(s, d)
    edges_b = _sorted_edges(d, s)
    dsel = jnp.zeros((R4, HC), jnp.float32)
    for h in range(H):
        dsel = dsel.at[HC + h, h * C:(h + 1) * C].set(1.0)

    accf = _direction(x[:, jnp.array([0, 1, 3])], edges_f,
                      params['f1'], params['f2'], params['f3'], dsel)
    accb = _direction(x[:, jnp.array([0, 2, 4])], edges_b,
                      params['b1'], params['b2'], params['b3'], dsel)

    hn, pacc = _run_pool(accf, accb,
                         params['f3']['b'].reshape(1, C),
                         params['b3']['b'].reshape(1, C), batch)
    g_pool = pacc[:, :2 * C] / jnp.clip(pacc[:, 2 * C:2 * C + 1], 1.0)
    return hn, g_pool
